# baseline (speedup 1.0000x reference)
import functools

import jax
import jax.numpy as jnp
from jax import lax
from jax.experimental import pallas as pl
from jax.experimental.pallas import tpu as pltpu
from jax.experimental.pallas import tpu_sc as plsc

_M = 128
_NC = 2
_NS = 16
_NW = _NC * _NS


def _gate_body(x_ref, w_ref, b_ref, wout_ref, iout_ref):
    e = w_ref.shape[0]
    xx = x_ref[...]
    logits = lax.dot_general(xx, w_ref[...], (((1,), (1,)), ((), ())),
                             preferred_element_type=jnp.float32)
    logits = logits + b_ref[...]
    m = jnp.max(logits, axis=-1, keepdims=True)
    p = jnp.exp(logits - m)
    probs = p / jnp.sum(p, axis=-1, keepdims=True)
    eidx = lax.broadcasted_iota(jnp.int32, probs.shape, 1)
    m1 = jnp.max(probs, axis=-1, keepdims=True)
    i1 = jnp.min(jnp.where(probs == m1, eidx, e), axis=-1, keepdims=True)
    masked = jnp.where(eidx == i1, -jnp.inf, probs)
    m2 = jnp.max(masked, axis=-1, keepdims=True)
    i2 = jnp.min(jnp.where(masked == m2, eidx, e), axis=-1, keepdims=True)
    wout_ref[...] = jnp.concatenate([m1, m2], axis=-1)
    iout_ref[...] = jnp.concatenate([i1, i2], axis=-1)


def _gate(xf, gate_w, gate_b):
    n, d = xf.shape
    e = gate_w.shape[0]
    return pl.pallas_call(
        _gate_body,
        out_shape=[jax.ShapeDtypeStruct((n, 2), jnp.float32),
                   jax.ShapeDtypeStruct((n, 2), jnp.int32)],
    )(xf, gate_w, gate_b.reshape(1, e))


def _gmm_body(be_ref, nb_ref, xg_ref, w1_ref, b1_ref, w2_ref, b2_ref, y_ref,
              w1b_ref, w2b_ref):
    i = pl.program_id(0)
    fresh = jnp.logical_or(i == 0, be_ref[i] != be_ref[jnp.maximum(i - 1, 0)])

    @pl.when(jnp.logical_and(i < nb_ref[0], fresh))
    def _():
        w1b_ref[...] = w1_ref[0].astype(jnp.bfloat16)
        w2b_ref[...] = w2_ref[0].astype(jnp.bfloat16)

    @pl.when(i < nb_ref[0])
    def _():
        xb = xg_ref[...].astype(jnp.bfloat16)
        h = lax.dot_general(xb, w1b_ref[...], (((1,), (1,)), ((), ())),
                            preferred_element_type=jnp.float32)
        h = jnp.maximum(h + b1_ref[0], 0.0).astype(jnp.bfloat16)
        y = lax.dot_general(h, w2b_ref[...], (((1,), (1,)), ((), ())),
                            preferred_element_type=jnp.float32)
        y_ref[...] = y + b2_ref[0]


def _gmm(blk_e, nblk, xg, W1, b1, W2, b2, g, p):
    e, h, d = W1.shape
    o = W2.shape[1]
    grid_spec = pltpu.PrefetchScalarGridSpec(
        num_scalar_prefetch=2,
        grid=(g,),
        in_specs=[
            pl.BlockSpec((_M, d), lambda i, be, nb: (i, 0)),
            pl.BlockSpec((1, h, d), lambda i, be, nb: (be[i], 0, 0)),
            pl.BlockSpec((1, 1, h), lambda i, be, nb: (be[i], 0, 0)),
            pl.BlockSpec((1, o, h), lambda i, be, nb: (be[i], 0, 0)),
            pl.BlockSpec((1, 1, o), lambda i, be, nb: (be[i], 0, 0)),
        ],
        out_specs=pl.BlockSpec((_M, o), lambda i, be, nb: (i, 0)),
        scratch_shapes=[pltpu.VMEM((h, d), jnp.bfloat16),
                        pltpu.VMEM((o, h), jnp.bfloat16)],
    )
    return pl.pallas_call(
        _gmm_body,
        grid_spec=grid_spec,
        out_shape=jax.ShapeDtypeStruct((p, o), jnp.float32),
        compiler_params=pltpu.CompilerParams(
            dimension_semantics=("arbitrary",)),
    )(blk_e, nblk, xg, W1, b1.reshape(e, 1, h), W2, b2.reshape(e, 1, o))


def _sc_dispatch(xf, dst_a3, dst_b3, p, d):
    n = xf.shape[0]
    tw = n // _NW
    mesh = plsc.VectorSubcoreMesh(core_axis_name="c", subcore_axis_name="s",
                                  num_cores=_NC, num_subcores=_NS)

    @functools.partial(
        pl.kernel,
        out_type=jax.ShapeDtypeStruct((p, d), jnp.float32),
        mesh=mesh,
        scratch_types=[pltpu.VMEM((1, tw), jnp.int32),
                       pltpu.VMEM((1, tw), jnp.int32),
                       pltpu.VMEM((tw, d), jnp.float32),
                       pltpu.SemaphoreType.DMA,
                       pltpu.SemaphoreType.DMA,
                       pltpu.SemaphoreType.DMA],
    )
    def disp_k(x_hbm, da_hbm, db_hbm, xg_hbm, ia_v, ib_v, v, sr, sa, sb):
        c = lax.axis_index("c")
        s = lax.axis_index("s")
        wid = c * _NS + s
        base = wid * tw
        cp = pltpu.async_copy(x_hbm.at[pl.ds(base, tw)], v, sr)
        pltpu.sync_copy(da_hbm.at[wid], ia_v)
        pltpu.sync_copy(db_hbm.at[wid], ib_v)
        cp.wait()
        ca = pltpu.async_copy(v, xg_hbm.at[ia_v.at[0]], sa)
        cb = pltpu.async_copy(v, xg_hbm.at[ib_v.at[0]], sb)
        ca.wait()
        cb.wait()

    return disp_k(xf, dst_a3, dst_b3)


def _sc_pairgather(y, dst_a3, dst_b3, n, o):
    tw = n // _NW
    half = tw // 2
    mesh = plsc.VectorSubcoreMesh(core_axis_name="c", subcore_axis_name="s",
                                  num_cores=_NC, num_subcores=_NS)

    @functools.partial(
        pl.kernel,
        out_type=[jax.ShapeDtypeStruct((n, o), jnp.float32),
                  jax.ShapeDtypeStruct((n, o), jnp.float32)],
        mesh=mesh,
        scratch_types=[pltpu.VMEM((1, tw), jnp.int32),
                       pltpu.VMEM((1, tw), jnp.int32),
                       pltpu.VMEM((half, o), jnp.float32),
                       pltpu.VMEM((half, o), jnp.float32),
                       pltpu.SemaphoreType.DMA,
                       pltpu.SemaphoreType.DMA,
                       pltpu.SemaphoreType.DMA,
                       pltpu.SemaphoreType.DMA],
    )
    def pg_k(y_hbm, da_hbm, db_hbm, ya_hbm, yb_hbm,
             ia_v, ib_v, va, vb, sa, sb, swa, swb):
        c = lax.axis_index("c")
        s = lax.axis_index("s")
        wid = c * _NS + s
        base = wid * tw
        pltpu.sync_copy(da_hbm.at[wid], ia_v)
        pltpu.sync_copy(db_hbm.at[wid], ib_v)
        ca = pltpu.async_copy(y_hbm.at[ia_v.at[0, pl.ds(0, half)]], va, sa)
        cb = pltpu.async_copy(y_hbm.at[ib_v.at[0, pl.ds(0, half)]], vb, sb)
        ca.wait()
        wa = pltpu.async_copy(va, ya_hbm.at[pl.ds(base, half)], swa)
        cb.wait()
        wb = pltpu.async_copy(vb, yb_hbm.at[pl.ds(base, half)], swb)
        wa.wait()
        ca2 = pltpu.async_copy(y_hbm.at[ia_v.at[0, pl.ds(half, half)]], va, sa)
        wb.wait()
        cb2 = pltpu.async_copy(y_hbm.at[ib_v.at[0, pl.ds(half, half)]], vb, sb)
        ca2.wait()
        wa2 = pltpu.async_copy(va, ya_hbm.at[pl.ds(base + half, half)], swa)
        cb2.wait()
        wb2 = pltpu.async_copy(vb, yb_hbm.at[pl.ds(base + half, half)], swb)
        wa2.wait()
        wb2.wait()

    return pg_k(y, dst_a3, dst_b3)


def _combine_body(ya_ref, yb_ref, wa_ref, wb_ref, o_ref):
    o_ref[...] = wa_ref[...] * ya_ref[...] + wb_ref[...] * yb_ref[...]


def _combine(ya, yb, wa, wb, n, o):
    blk = 256
    return pl.pallas_call(
        _combine_body,
        grid=(n // blk,),
        in_specs=[pl.BlockSpec((blk, o), lambda i: (i, 0)),
                  pl.BlockSpec((blk, o), lambda i: (i, 0)),
                  pl.BlockSpec((blk, 1), lambda i: (i, 0)),
                  pl.BlockSpec((blk, 1), lambda i: (i, 0))],
        out_specs=pl.BlockSpec((blk, o), lambda i: (i, 0)),
        out_shape=jax.ShapeDtypeStruct((n, o), jnp.float32),
    )(ya, yb, wa, wb)


def _routing(i01, e, n):
    npairs = i01.size
    e_flat = i01.reshape(-1)
    onehot = (e_flat[:, None] == jnp.arange(e, dtype=e_flat.dtype)[None, :])
    onehot = onehot.astype(jnp.int32)
    ranks = jnp.cumsum(onehot, axis=0) - onehot
    rank = jnp.sum(ranks * onehot, axis=-1)
    counts = jnp.sum(onehot, axis=0)
    padded = ((counts + _M - 1) // _M) * _M
    pad_off = jnp.concatenate(
        [jnp.zeros((1,), jnp.int32), jnp.cumsum(padded)[:-1].astype(jnp.int32)])
    g = npairs // _M + e
    p = g * _M
    dst = rank + jnp.sum(onehot * pad_off[None, :], axis=-1)
    total = jnp.sum(padded)
    ends = pad_off + padded
    bstart = jnp.arange(g, dtype=jnp.int32) * _M
    blk_e = jnp.sum((bstart[:, None] >= ends[None, :]).astype(jnp.int32),
                    axis=-1)
    last_e = jnp.max(jnp.where(padded > 0, jnp.arange(e, dtype=jnp.int32), 0))
    blk_e = jnp.where(bstart < total, blk_e, last_e).astype(jnp.int32)
    nblk = (total // _M).astype(jnp.int32).reshape(1)
    dst2 = dst.reshape(n, 2)
    dst_a3 = dst2[:, 0].reshape(_NW, 1, n // _NW)
    dst_b3 = dst2[:, 1].reshape(_NW, 1, n // _NW)
    return dst_a3, dst_b3, blk_e, nblk, g, p


def kernel(x, gate_w, gate_b, W1, b1, W2, b2):
    bsz, seq, d = x.shape
    n = bsz * seq
    e, h, _ = W1.shape
    o = W2.shape[1]
    xf = x.reshape(n, d)

    w01, i01 = _gate(xf, gate_w, gate_b)
    dst_a3, dst_b3, blk_e, nblk, g, p = _routing(i01, e, n)
    xg = _sc_dispatch(xf, dst_a3, dst_b3, p, d)
    y = _gmm(blk_e, nblk, xg, W1, b1, W2, b2, g, p)
    ya, yb = _sc_pairgather(y, dst_a3, dst_b3, n, o)
    out = _combine(ya, yb, w01[:, 0:1], w01[:, 1:2], n, o)
    return out.reshape(bsz, seq, o)


# speedup vs baseline: 1.4804x; 1.4804x over previous
import functools

import jax
import jax.numpy as jnp
from jax import lax
from jax.experimental import pallas as pl
from jax.experimental.pallas import tpu as pltpu
from jax.experimental.pallas import tpu_sc as plsc

_M = 512
_NC = 2
_NS = 16
_NW = _NC * _NS


def _gate_body(x_ref, w_ref, b_ref, wout_ref, da_ref, db_ref, be_ref, nb_ref):
    e = w_ref.shape[0]
    n = x_ref.shape[0]
    g = be_ref.shape[0]
    m = float(_M)
    lg = _M.bit_length() - 1
    xx = x_ref[...]
    logits = lax.dot_general(xx, w_ref[...], (((1,), (1,)), ((), ())),
                             preferred_element_type=jnp.float32)
    logits = logits + b_ref[...]
    mx = jnp.max(logits, axis=-1, keepdims=True)
    pe = jnp.exp(logits - mx)
    probs = pe / jnp.sum(pe, axis=-1, keepdims=True)
    eidx = lax.broadcasted_iota(jnp.int32, probs.shape, 1)
    m1 = jnp.max(probs, axis=-1, keepdims=True)
    i1 = jnp.min(jnp.where(probs == m1, eidx, e), axis=-1, keepdims=True)
    masked = jnp.where(eidx == i1, -jnp.inf, probs)
    m2 = jnp.max(masked, axis=-1, keepdims=True)
    i2 = jnp.min(jnp.where(masked == m2, eidx, e), axis=-1, keepdims=True)
    wout_ref[...] = jnp.concatenate([m1, m2], axis=-1)

    oa = (eidx == i1).astype(jnp.float32)
    ob = (eidx == i2).astype(jnp.float32)
    s = oa + ob
    c = s
    k = 1
    while k < n:
        c = c + jnp.concatenate(
            [jnp.zeros((k, e), jnp.float32), c[:n - k, :]], axis=0)
        k *= 2
    excl = c - s
    counts_i = c[n - 1:n, :].astype(jnp.int32)
    padded_i = ((counts_i + (_M - 1)) >> lg) << lg
    padded = padded_i.astype(jnp.float32)
    pc = padded
    k = 1
    while k < e:
        pc = pc + jnp.concatenate(
            [jnp.zeros((1, k), jnp.float32), pc[:, :e - k]], axis=1)
        k *= 2
    pad_off = pc - padded
    total = pc[:, e - 1:e]
    base = excl + pad_off
    da_ref[...] = jnp.sum(oa * base, axis=-1, keepdims=True).astype(jnp.int32)
    db_ref[...] = jnp.sum(ob * base, axis=-1, keepdims=True).astype(jnp.int32)
    ends = pad_off + padded
    bstart = lax.broadcasted_iota(jnp.int32, (g, 1), 0).astype(jnp.float32) * m
    blk = jnp.sum((bstart >= ends).astype(jnp.float32), axis=-1, keepdims=True)
    eflt = lax.broadcasted_iota(jnp.int32, (1, e), 1).astype(jnp.float32)
    last_e = jnp.max(jnp.where(padded > 0, eflt, 0.0), axis=-1, keepdims=True)
    blk = jnp.where(bstart < total, blk, last_e)
    be_ref[...] = blk.astype(jnp.int32)
    nb_ref[...] = (total.astype(jnp.int32)) >> lg


def _gate(xf, gate_w, gate_b, g):
    n, d = xf.shape
    e = gate_w.shape[0]
    return pl.pallas_call(
        _gate_body,
        out_shape=[jax.ShapeDtypeStruct((n, 2), jnp.float32),
                   jax.ShapeDtypeStruct((n, 1), jnp.int32),
                   jax.ShapeDtypeStruct((n, 1), jnp.int32),
                   jax.ShapeDtypeStruct((g, 1), jnp.int32),
                   jax.ShapeDtypeStruct((1, 1), jnp.int32)],
    )(xf, gate_w, gate_b.reshape(1, e))


def _gmm_body(be_ref, nb_ref, xg_ref, w1_ref, b1_ref, w2_ref, b2_ref, y_ref,
              w1b_ref, w2b_ref):
    i = pl.program_id(0)
    fresh = jnp.logical_or(i == 0, be_ref[i] != be_ref[jnp.maximum(i - 1, 0)])

    @pl.when(jnp.logical_and(i < nb_ref[0], fresh))
    def _():
        w1b_ref[...] = w1_ref[0].astype(jnp.bfloat16)
        w2b_ref[...] = w2_ref[0].astype(jnp.bfloat16)

    @pl.when(i < nb_ref[0])
    def _():
        xb = xg_ref[...].astype(jnp.bfloat16)
        h = lax.dot_general(xb, w1b_ref[...], (((1,), (1,)), ((), ())),
                            preferred_element_type=jnp.float32)
        h = jnp.maximum(h + b1_ref[0], 0.0).astype(jnp.bfloat16)
        y = lax.dot_general(h, w2b_ref[...], (((1,), (1,)), ((), ())),
                            preferred_element_type=jnp.float32)
        y_ref[...] = y + b2_ref[0]


def _gmm(blk_e, nblk, xg, W1, b1, W2, b2, g, p):
    e, h, d = W1.shape
    o = W2.shape[1]
    grid_spec = pltpu.PrefetchScalarGridSpec(
        num_scalar_prefetch=2,
        grid=(g,),
        in_specs=[
            pl.BlockSpec((_M, d), lambda i, be, nb: (i, 0)),
            pl.BlockSpec((1, h, d), lambda i, be, nb: (be[i], 0, 0)),
            pl.BlockSpec((1, 1, h), lambda i, be, nb: (be[i], 0, 0)),
            pl.BlockSpec((1, o, h), lambda i, be, nb: (be[i], 0, 0)),
            pl.BlockSpec((1, 1, o), lambda i, be, nb: (be[i], 0, 0)),
        ],
        out_specs=pl.BlockSpec((_M, o), lambda i, be, nb: (i, 0)),
        scratch_shapes=[pltpu.VMEM((h, d), jnp.bfloat16),
                        pltpu.VMEM((o, h), jnp.bfloat16)],
    )
    return pl.pallas_call(
        _gmm_body,
        grid_spec=grid_spec,
        out_shape=jax.ShapeDtypeStruct((p, o), jnp.float32),
        compiler_params=pltpu.CompilerParams(
            dimension_semantics=("arbitrary",),
            vmem_limit_bytes=100 * 1024 * 1024),
    )(blk_e, nblk, xg, W1, b1.reshape(e, 1, h), W2, b2.reshape(e, 1, o))


def _sc_dispatch(xf, dst_a3, dst_b3, p, d):
    n = xf.shape[0]
    tw = n // _NW
    mesh = plsc.VectorSubcoreMesh(core_axis_name="c", subcore_axis_name="s",
                                  num_cores=_NC, num_subcores=_NS)

    @functools.partial(
        pl.kernel,
        out_type=jax.ShapeDtypeStruct((p, d), jnp.float32),
        mesh=mesh,
        scratch_types=[pltpu.VMEM((1, tw), jnp.int32),
                       pltpu.VMEM((1, tw), jnp.int32),
                       pltpu.VMEM((tw, d), jnp.float32),
                       pltpu.SemaphoreType.DMA,
                       pltpu.SemaphoreType.DMA,
                       pltpu.SemaphoreType.DMA],
    )
    def disp_k(x_hbm, da_hbm, db_hbm, xg_hbm, ia_v, ib_v, v, sr, sa, sb):
        c = lax.axis_index("c")
        s = lax.axis_index("s")
        wid = c * _NS + s
        base = wid * tw
        cp = pltpu.async_copy(x_hbm.at[pl.ds(base, tw)], v, sr)
        pltpu.sync_copy(da_hbm.at[wid], ia_v)
        pltpu.sync_copy(db_hbm.at[wid], ib_v)
        cp.wait()
        ca = pltpu.async_copy(v, xg_hbm.at[ia_v.at[0]], sa)
        cb = pltpu.async_copy(v, xg_hbm.at[ib_v.at[0]], sb)
        ca.wait()
        cb.wait()

    return disp_k(xf, dst_a3, dst_b3)


def _sc_pairgather(y, dst_a3, dst_b3, n, o):
    tw = n // _NW
    half = tw // 2
    mesh = plsc.VectorSubcoreMesh(core_axis_name="c", subcore_axis_name="s",
                                  num_cores=_NC, num_subcores=_NS)

    @functools.partial(
        pl.kernel,
        out_type=[jax.ShapeDtypeStruct((n, o), jnp.float32),
                  jax.ShapeDtypeStruct((n, o), jnp.float32)],
        mesh=mesh,
        scratch_types=[pltpu.VMEM((1, tw), jnp.int32),
                       pltpu.VMEM((1, tw), jnp.int32),
                       pltpu.VMEM((half, o), jnp.float32),
                       pltpu.VMEM((half, o), jnp.float32),
                       pltpu.SemaphoreType.DMA,
                       pltpu.SemaphoreType.DMA,
                       pltpu.SemaphoreType.DMA,
                       pltpu.SemaphoreType.DMA],
    )
    def pg_k(y_hbm, da_hbm, db_hbm, ya_hbm, yb_hbm,
             ia_v, ib_v, va, vb, sa, sb, swa, swb):
        c = lax.axis_index("c")
        s = lax.axis_index("s")
        wid = c * _NS + s
        base = wid * tw
        pltpu.sync_copy(da_hbm.at[wid], ia_v)
        pltpu.sync_copy(db_hbm.at[wid], ib_v)
        ca = pltpu.async_copy(y_hbm.at[ia_v.at[0, pl.ds(0, half)]], va, sa)
        cb = pltpu.async_copy(y_hbm.at[ib_v.at[0, pl.ds(0, half)]], vb, sb)
        ca.wait()
        wa = pltpu.async_copy(va, ya_hbm.at[pl.ds(base, half)], swa)
        cb.wait()
        wb = pltpu.async_copy(vb, yb_hbm.at[pl.ds(base, half)], swb)
        wa.wait()
        ca2 = pltpu.async_copy(y_hbm.at[ia_v.at[0, pl.ds(half, half)]], va, sa)
        wb.wait()
        cb2 = pltpu.async_copy(y_hbm.at[ib_v.at[0, pl.ds(half, half)]], vb, sb)
        ca2.wait()
        wa2 = pltpu.async_copy(va, ya_hbm.at[pl.ds(base + half, half)], swa)
        cb2.wait()
        wb2 = pltpu.async_copy(vb, yb_hbm.at[pl.ds(base + half, half)], swb)
        wa2.wait()
        wb2.wait()

    return pg_k(y, dst_a3, dst_b3)


def _combine_body(ya_ref, yb_ref, wa_ref, wb_ref, o_ref):
    o_ref[...] = wa_ref[...] * ya_ref[...] + wb_ref[...] * yb_ref[...]


def _combine(ya, yb, wa, wb, n, o):
    blk = 256
    return pl.pallas_call(
        _combine_body,
        grid=(n // blk,),
        in_specs=[pl.BlockSpec((blk, o), lambda i: (i, 0)),
                  pl.BlockSpec((blk, o), lambda i: (i, 0)),
                  pl.BlockSpec((blk, 1), lambda i: (i, 0)),
                  pl.BlockSpec((blk, 1), lambda i: (i, 0))],
        out_specs=pl.BlockSpec((blk, o), lambda i: (i, 0)),
        out_shape=jax.ShapeDtypeStruct((n, o), jnp.float32),
    )(ya, yb, wa, wb)


def kernel(x, gate_w, gate_b, W1, b1, W2, b2):
    bsz, seq, d = x.shape
    n = bsz * seq
    e, h, _ = W1.shape
    o = W2.shape[1]
    g = 2 * n // _M + e
    p = g * _M
    xf = x.reshape(n, d)

    w01, da, db, be2, nb2 = _gate(xf, gate_w, gate_b, g)
    dst_a3 = da.reshape(_NW, 1, n // _NW)
    dst_b3 = db.reshape(_NW, 1, n // _NW)
    blk_e = be2.reshape(g)
    nblk = nb2.reshape(1)
    xg = _sc_dispatch(xf, dst_a3, dst_b3, p, d)
    y = _gmm(blk_e, nblk, xg, W1, b1, W2, b2, g, p)
    ya, yb = _sc_pairgather(y, dst_a3, dst_b3, n, o)
    out = _combine(ya, yb, w01[:, 0:1], w01[:, 1:2], n, o)
    return out.reshape(bsz, seq, o)
